# baseline (speedup 1.0000x reference)
import functools

import jax
import jax.numpy as jnp
from jax import lax
from jax.experimental import pallas as pl
from jax.experimental.pallas import tpu as pltpu
from jax.experimental.pallas import tpu_sc as plsc

N_NODES = 10000
D = 128
N_PAD = 10240
NC = 2
NS = 16
NW = NC * NS
K = 128
NB = 2
C = 64
CP = C + 2 * NB
EPW = CP * K
ROWS_PER_TILE = N_PAD // NS
RING = 2 * NB
BM = 1024
E_TC = 65536
E_SC = NW * C * K
EB = 2048
EPW_H = 10240

_SC_MESH = plsc.VectorSubcoreMesh(core_axis_name="c", subcore_axis_name="s")


@functools.partial(
    pl.kernel,
    out_type=jax.ShapeDtypeStruct((NW, N_PAD), jnp.float32),
    mesh=_SC_MESH,
    scratch_types=[
        pltpu.VMEM((EPW_H,), jnp.int32),
        pltpu.VMEM((N_PAD,), jnp.float32),
    ],
    compiler_params=pltpu.CompilerParams(needs_layout_passes=False),
)
def _sc_degree_hist(dst_hbm, out_hbm, dst_v, hist_v):
    cid = lax.axis_index("c")
    sid = lax.axis_index("s")
    wid = sid * NC + cid
    pltpu.sync_copy(dst_hbm.at[wid], dst_v)

    zeros16 = jnp.zeros((16,), jnp.float32)

    def zero_body(i, carry):
        hist_v[pl.ds(i * 16, 16)] = zeros16
        return carry

    lax.fori_loop(0, N_PAD // 16, zero_body, 0)

    ones16 = jnp.ones((16,), jnp.float32)

    def hist_body(i, carry):
        idx = dst_v[pl.ds(i * 16, 16)]
        plsc.addupdate_scatter(hist_v, [idx], ones16)
        return carry

    lax.fori_loop(0, EPW_H // 16, hist_body, 0)
    pltpu.sync_copy(hist_v, out_hbm.at[wid])


@functools.partial(
    pl.kernel,
    out_type=jax.ShapeDtypeStruct((NC, N_PAD, D), jnp.float32),
    mesh=_SC_MESH,
    scratch_types=[
        pltpu.VMEM((RING, 2, K), jnp.int32),
        pltpu.VMEM((NB, K, D), jnp.float32),
        pltpu.VMEM_SHARED((N_PAD, D), jnp.float32),
    ] + [pltpu.SemaphoreType.DMA] * (3 * NB),
)
def _sc_edge_agg(hs_hbm, idx_hbm, out_hbm, idx_v, rows_v, acc_sh, *sems):
    cid = lax.axis_index("c")
    sid = lax.axis_index("s")
    wid = sid * NC + cid
    gsems = sems[0:NB]
    isems = sems[NB:2 * NB]
    ssems = sems[2 * NB:3 * NB]

    zeros16 = jnp.zeros((16,), jnp.float32)

    def zero_body(i, carry):
        for j in range(D // 16):
            rows_v[0, i, pl.ds(j * 16, 16)] = zeros16
        return carry

    lax.fori_loop(0, K, zero_body, 0)
    base = sid * ROWS_PER_TILE
    for j in range(ROWS_PER_TILE // K):
        pltpu.sync_copy(rows_v.at[0], acc_sh.at[pl.ds(base + j * K, K)])
    plsc.subcore_barrier()

    for s in range(NB):
        pltpu.sync_copy(idx_hbm.at[wid, s], idx_v.at[s])
    for j in range(NB):
        pltpu.async_copy(idx_hbm.at[wid, NB + j], idx_v.at[NB + j], isems[j])
    for b in range(NB):
        pltpu.async_copy(hs_hbm.at[idx_v.at[b, 0]], rows_v.at[b], gsems[b])

    def chunk_body(g, carry):
        for b in range(NB):
            cc = g * NB + b
            s0 = lax.rem(cc, RING)
            sg = lax.rem(cc + NB, RING)
            pltpu.make_async_copy(
                hs_hbm.at[idx_v.at[s0, 0]], rows_v.at[b], gsems[b]).wait()
            pltpu.async_copy(rows_v.at[b], acc_sh.at[idx_v.at[s0, 1]],
                             ssems[b], add=True)
            pltpu.make_async_copy(
                idx_hbm.at[wid, cc + NB], idx_v.at[sg], isems[b]).wait()
            pltpu.make_async_copy(rows_v.at[b], acc_sh.at[idx_v.at[s0, 1]],
                                  ssems[b]).wait()
            pltpu.async_copy(idx_hbm.at[wid, cc + 2 * NB], idx_v.at[s0],
                             isems[b])
            pltpu.async_copy(
                hs_hbm.at[idx_v.at[sg, 0]], rows_v.at[b], gsems[b])
        return carry

    lax.fori_loop(0, C // NB, chunk_body, 0)

    for b in range(NB):
        pltpu.make_async_copy(
            hs_hbm.at[idx_v.at[b, 0]], rows_v.at[b], gsems[b]).wait()
        pltpu.make_async_copy(
            idx_hbm.at[wid, C + NB + b], idx_v.at[NB + b], isems[b]).wait()

    plsc.subcore_barrier()

    for j in range(ROWS_PER_TILE // K):
        r0 = base + j * K
        pltpu.sync_copy(acc_sh.at[pl.ds(r0, K)], rows_v.at[0])
        pltpu.sync_copy(rows_v.at[0], out_hbm.at[cid, pl.ds(r0, K)])


def _tc_agg_body(idx_ref, hs_ref, out_ref, acc_ref):
    step = pl.program_id(0)

    @pl.when(step == 0)
    def _():
        acc_ref[...] = jnp.zeros_like(acc_ref)

    def body(e, carry):
        s = idx_ref[0, e]
        d = idx_ref[1, e]
        acc_ref[pl.ds(d, 1), :] += hs_ref[pl.ds(s, 1), :]
        return carry

    lax.fori_loop(0, EB, body, 0)

    @pl.when(step == E_TC // EB - 1)
    def _():
        out_ref[...] = acc_ref[...]


def _tc_agg(idx_tc, hs):
    return pl.pallas_call(
        _tc_agg_body,
        grid=(E_TC // EB,),
        in_specs=[
            pl.BlockSpec((2, EB), lambda i: (0, i), memory_space=pltpu.SMEM),
            pl.BlockSpec((N_PAD, D), lambda i: (0, 0)),
        ],
        out_specs=pl.BlockSpec((N_PAD, D), lambda i: (0, 0)),
        out_shape=jax.ShapeDtypeStruct((N_PAD, D), jnp.float32),
        scratch_shapes=[pltpu.VMEM((N_PAD, D), jnp.float32)],
        compiler_params=pltpu.CompilerParams(
            dimension_semantics=("arbitrary",)),
    )(idx_tc, hs)


def _tc_pre_body(hist_ref, x_ref, w1_ref, h1_ref, hs1_ref, dis_ref, inv_ref):
    deg = jnp.sum(hist_ref[...], axis=0) + 1.0
    dis = lax.rsqrt(deg)
    inv = 1.0 / deg
    h1 = jnp.dot(x_ref[...], w1_ref[...], preferred_element_type=jnp.float32)
    h1_ref[...] = h1
    hs1_ref[...] = h1 * dis[:, None]
    dis_ref[...] = dis
    inv_ref[...] = inv


def _tc_pre(hists, x_p, w1):
    return pl.pallas_call(
        _tc_pre_body,
        grid=(N_PAD // BM,),
        in_specs=[
            pl.BlockSpec((NW, BM), lambda i: (0, i)),
            pl.BlockSpec((BM, D), lambda i: (i, 0)),
            pl.BlockSpec((D, D), lambda i: (0, 0)),
        ],
        out_specs=[
            pl.BlockSpec((BM, D), lambda i: (i, 0)),
            pl.BlockSpec((BM, D), lambda i: (i, 0)),
            pl.BlockSpec((BM,), lambda i: (i,)),
            pl.BlockSpec((BM,), lambda i: (i,)),
        ],
        out_shape=[
            jax.ShapeDtypeStruct((N_PAD, D), jnp.float32),
            jax.ShapeDtypeStruct((N_PAD, D), jnp.float32),
            jax.ShapeDtypeStruct((N_PAD,), jnp.float32),
            jax.ShapeDtypeStruct((N_PAD,), jnp.float32),
        ],
    )(hists, x_p, w1)


def _tc_mid_body(p_ref, pt_ref, h1_ref, dis_ref, inv_ref, b1_ref, w2_ref,
                 h2_ref, hs2_ref):
    agg = p_ref[0] + p_ref[1] + pt_ref[...]
    dis = dis_ref[...]
    a1 = (agg * dis[:, None] + h1_ref[...] * inv_ref[...][:, None]
          + b1_ref[...][None, :])
    h1o = jnp.maximum(a1, 0.0)
    h2 = jnp.dot(h1o, w2_ref[...], preferred_element_type=jnp.float32)
    h2_ref[...] = h2
    hs2_ref[...] = h2 * dis[:, None]


def _tc_mid(p1, pt1, h1, dis, inv, b1, w2):
    return pl.pallas_call(
        _tc_mid_body,
        grid=(N_PAD // BM,),
        in_specs=[
            pl.BlockSpec((NC, BM, D), lambda i: (0, i, 0)),
            pl.BlockSpec((BM, D), lambda i: (i, 0)),
            pl.BlockSpec((BM, D), lambda i: (i, 0)),
            pl.BlockSpec((BM,), lambda i: (i,)),
            pl.BlockSpec((BM,), lambda i: (i,)),
            pl.BlockSpec((D,), lambda i: (0,)),
            pl.BlockSpec((D, D), lambda i: (0, 0)),
        ],
        out_specs=[
            pl.BlockSpec((BM, D), lambda i: (i, 0)),
            pl.BlockSpec((BM, D), lambda i: (i, 0)),
        ],
        out_shape=[
            jax.ShapeDtypeStruct((N_PAD, D), jnp.float32),
            jax.ShapeDtypeStruct((N_PAD, D), jnp.float32),
        ],
    )(p1, pt1, h1, dis, inv, b1, w2)


def _tc_post_body(p_ref, pt_ref, h2_ref, dis_ref, inv_ref, b2_ref, out_ref):
    agg = p_ref[0] + p_ref[1] + pt_ref[...]
    out_ref[...] = (agg * dis_ref[...][:, None]
                    + h2_ref[...] * inv_ref[...][:, None]
                    + b2_ref[...][None, :])


def _tc_post(p2, pt2, h2, dis, inv, b2):
    return pl.pallas_call(
        _tc_post_body,
        grid=(N_PAD // BM,),
        in_specs=[
            pl.BlockSpec((NC, BM, D), lambda i: (0, i, 0)),
            pl.BlockSpec((BM, D), lambda i: (i, 0)),
            pl.BlockSpec((BM, D), lambda i: (i, 0)),
            pl.BlockSpec((BM,), lambda i: (i,)),
            pl.BlockSpec((BM,), lambda i: (i,)),
            pl.BlockSpec((D,), lambda i: (0,)),
        ],
        out_specs=pl.BlockSpec((BM, D), lambda i: (i, 0)),
        out_shape=jax.ShapeDtypeStruct((N_PAD, D), jnp.float32),
    )(p2, pt2, h2, dis, inv, b2)


def kernel(x, edge_index, W1, b1, W2, b2):
    e = edge_index.shape[1]
    src = edge_index[0]
    dst = edge_index[1]

    e_sc = e - E_TC
    idx_tc = edge_index[:, e_sc:]

    pad_real = E_SC - e_sc
    padv = jnp.full((pad_real,), N_NODES, jnp.int32)
    src_r = jnp.concatenate([src[:e_sc], padv]).reshape(NW, C, K)
    dst_r = jnp.concatenate([dst[:e_sc], padv]).reshape(NW, C, K)
    pad_chunks = jnp.full((NW, CP - C, K), N_NODES, jnp.int32)
    src_p = jnp.concatenate([src_r, pad_chunks], axis=1)
    dst_p = jnp.concatenate([dst_r, pad_chunks], axis=1)
    idx_comb = jnp.stack([src_p, dst_p], axis=2)

    pad_h = NW * EPW_H - e
    dst_h = jnp.concatenate(
        [dst, jnp.full((pad_h,), N_NODES, jnp.int32)]).reshape(NW, EPW_H)

    x_p = jnp.zeros((N_PAD, D), jnp.float32).at[:N_NODES].set(x)

    hists = _sc_degree_hist(dst_h)
    h1, hs1, dis, inv_deg = _tc_pre(hists, x_p, W1)
    p1 = _sc_edge_agg(hs1, idx_comb)
    pt1 = _tc_agg(idx_tc, hs1)
    h2, hs2 = _tc_mid(p1, pt1, h1, dis, inv_deg, b1, W2)
    p2 = _sc_edge_agg(hs2, idx_comb)
    pt2 = _tc_agg(idx_tc, hs2)
    out_p = _tc_post(p2, pt2, h2, dis, inv_deg, b2)
    return out_p[:N_NODES]


# speedup vs baseline: 1.0795x; 1.0795x over previous
import functools

import jax
import jax.numpy as jnp
from jax import lax
from jax.experimental import pallas as pl
from jax.experimental.pallas import tpu as pltpu
from jax.experimental.pallas import tpu_sc as plsc

N_NODES = 10000
D = 128
N_PAD = 10240
NC = 2
NS = 16
NW = NC * NS
K = 128
NB = 2
C = 80
CP = C + 2 * NB
ROWS_PER_TILE = N_PAD // NS
RING = 2 * NB
BM = 1024
E_SC = NW * C * K
EPW_H = 10240

_SC_MESH = plsc.VectorSubcoreMesh(core_axis_name="c", subcore_axis_name="s")


@functools.partial(
    pl.kernel,
    out_type=jax.ShapeDtypeStruct((NW, N_PAD), jnp.float32),
    mesh=_SC_MESH,
    scratch_types=[
        pltpu.VMEM((EPW_H,), jnp.int32),
        pltpu.VMEM((N_PAD,), jnp.float32),
    ],
    compiler_params=pltpu.CompilerParams(needs_layout_passes=False),
)
def _sc_degree_hist(dst_hbm, out_hbm, dst_v, hist_v):
    cid = lax.axis_index("c")
    sid = lax.axis_index("s")
    wid = sid * NC + cid
    pltpu.sync_copy(dst_hbm.at[wid], dst_v)

    zeros16 = jnp.zeros((16,), jnp.float32)

    def zero_body(i, carry):
        hist_v[pl.ds(i * 16, 16)] = zeros16
        return carry

    lax.fori_loop(0, N_PAD // 16, zero_body, 0)

    ones16 = jnp.ones((16,), jnp.float32)

    def hist_body(i, carry):
        idx = dst_v[pl.ds(i * 16, 16)]
        plsc.addupdate_scatter(hist_v, [idx], ones16)
        return carry

    lax.fori_loop(0, EPW_H // 16, hist_body, 0)
    pltpu.sync_copy(hist_v, out_hbm.at[wid])


@functools.partial(
    pl.kernel,
    out_type=jax.ShapeDtypeStruct((NC, N_PAD, D), jnp.float32),
    mesh=_SC_MESH,
    scratch_types=[
        pltpu.VMEM((RING, 2, K), jnp.int32),
        pltpu.VMEM((NB, K, D), jnp.float32),
        pltpu.VMEM_SHARED((N_PAD, D), jnp.float32),
    ] + [pltpu.SemaphoreType.DMA] * (3 * NB),
)
def _sc_edge_agg(hs_hbm, idx_hbm, out_hbm, idx_v, rows_v, acc_sh, *sems):
    cid = lax.axis_index("c")
    sid = lax.axis_index("s")
    wid = sid * NC + cid
    gsems = sems[0:NB]
    isems = sems[NB:2 * NB]
    ssems = sems[2 * NB:3 * NB]

    zeros16 = jnp.zeros((16,), jnp.float32)

    def zero_body(i, carry):
        for j in range(D // 16):
            rows_v[0, i, pl.ds(j * 16, 16)] = zeros16
        return carry

    lax.fori_loop(0, K, zero_body, 0)
    base = sid * ROWS_PER_TILE
    for j in range(ROWS_PER_TILE // K):
        pltpu.sync_copy(rows_v.at[0], acc_sh.at[pl.ds(base + j * K, K)])
    plsc.subcore_barrier()

    for s in range(NB):
        pltpu.sync_copy(idx_hbm.at[wid, s], idx_v.at[s])
    for j in range(NB):
        pltpu.async_copy(idx_hbm.at[wid, NB + j], idx_v.at[NB + j], isems[j])
    for b in range(NB):
        pltpu.async_copy(hs_hbm.at[idx_v.at[b, 0]], rows_v.at[b], gsems[b])

    def chunk_body(g, carry):
        for b in range(NB):
            cc = g * NB + b
            s0 = lax.rem(cc, RING)
            sg = lax.rem(cc + NB, RING)
            pltpu.make_async_copy(
                hs_hbm.at[idx_v.at[s0, 0]], rows_v.at[b], gsems[b]).wait()
            pltpu.async_copy(rows_v.at[b], acc_sh.at[idx_v.at[s0, 1]],
                             ssems[b], add=True)
            pltpu.make_async_copy(
                idx_hbm.at[wid, cc + NB], idx_v.at[sg], isems[b]).wait()
            pltpu.make_async_copy(rows_v.at[b], acc_sh.at[idx_v.at[s0, 1]],
                                  ssems[b]).wait()
            pltpu.async_copy(idx_hbm.at[wid, cc + 2 * NB], idx_v.at[s0],
                             isems[b])
            pltpu.async_copy(
                hs_hbm.at[idx_v.at[sg, 0]], rows_v.at[b], gsems[b])
        return carry

    lax.fori_loop(0, C // NB, chunk_body, 0)

    for b in range(NB):
        pltpu.make_async_copy(
            hs_hbm.at[idx_v.at[b, 0]], rows_v.at[b], gsems[b]).wait()
        pltpu.make_async_copy(
            idx_hbm.at[wid, C + NB + b], idx_v.at[NB + b], isems[b]).wait()

    plsc.subcore_barrier()

    for j in range(ROWS_PER_TILE // K):
        r0 = base + j * K
        pltpu.sync_copy(acc_sh.at[pl.ds(r0, K)], rows_v.at[0])
        pltpu.sync_copy(rows_v.at[0], out_hbm.at[cid, pl.ds(r0, K)])


def _tc_pre_body(hist_ref, x_ref, w1_ref, h1_ref, hs1_ref, dis_ref, inv_ref):
    deg = jnp.sum(hist_ref[...], axis=0) + 1.0
    dis = lax.rsqrt(deg)
    inv = 1.0 / deg
    h1 = jnp.dot(x_ref[...], w1_ref[...], preferred_element_type=jnp.float32)
    h1_ref[...] = h1
    hs1_ref[...] = h1 * dis[:, None]
    dis_ref[...] = dis
    inv_ref[...] = inv


def _tc_pre(hists, x_p, w1):
    return pl.pallas_call(
        _tc_pre_body,
        grid=(N_PAD // BM,),
        in_specs=[
            pl.BlockSpec((NW, BM), lambda i: (0, i)),
            pl.BlockSpec((BM, D), lambda i: (i, 0)),
            pl.BlockSpec((D, D), lambda i: (0, 0)),
        ],
        out_specs=[
            pl.BlockSpec((BM, D), lambda i: (i, 0)),
            pl.BlockSpec((BM, D), lambda i: (i, 0)),
            pl.BlockSpec((BM,), lambda i: (i,)),
            pl.BlockSpec((BM,), lambda i: (i,)),
        ],
        out_shape=[
            jax.ShapeDtypeStruct((N_PAD, D), jnp.float32),
            jax.ShapeDtypeStruct((N_PAD, D), jnp.float32),
            jax.ShapeDtypeStruct((N_PAD,), jnp.float32),
            jax.ShapeDtypeStruct((N_PAD,), jnp.float32),
        ],
    )(hists, x_p, w1)


def _tc_mid_body(p_ref, h1_ref, dis_ref, inv_ref, b1_ref, w2_ref,
                 h2_ref, hs2_ref):
    agg = p_ref[0] + p_ref[1]
    dis = dis_ref[...]
    a1 = (agg * dis[:, None] + h1_ref[...] * inv_ref[...][:, None]
          + b1_ref[...][None, :])
    h1o = jnp.maximum(a1, 0.0)
    h2 = jnp.dot(h1o, w2_ref[...], preferred_element_type=jnp.float32)
    h2_ref[...] = h2
    hs2_ref[...] = h2 * dis[:, None]


def _tc_mid(p1, h1, dis, inv, b1, w2):
    return pl.pallas_call(
        _tc_mid_body,
        grid=(N_PAD // BM,),
        in_specs=[
            pl.BlockSpec((NC, BM, D), lambda i: (0, i, 0)),
            pl.BlockSpec((BM, D), lambda i: (i, 0)),
            pl.BlockSpec((BM,), lambda i: (i,)),
            pl.BlockSpec((BM,), lambda i: (i,)),
            pl.BlockSpec((D,), lambda i: (0,)),
            pl.BlockSpec((D, D), lambda i: (0, 0)),
        ],
        out_specs=[
            pl.BlockSpec((BM, D), lambda i: (i, 0)),
            pl.BlockSpec((BM, D), lambda i: (i, 0)),
        ],
        out_shape=[
            jax.ShapeDtypeStruct((N_PAD, D), jnp.float32),
            jax.ShapeDtypeStruct((N_PAD, D), jnp.float32),
        ],
    )(p1, h1, dis, inv, b1, w2)


def _tc_post_body(p_ref, h2_ref, dis_ref, inv_ref, b2_ref, out_ref):
    agg = p_ref[0] + p_ref[1]
    out_ref[...] = (agg * dis_ref[...][:, None]
                    + h2_ref[...] * inv_ref[...][:, None]
                    + b2_ref[...][None, :])


def _tc_post(p2, h2, dis, inv, b2):
    return pl.pallas_call(
        _tc_post_body,
        grid=(N_PAD // BM,),
        in_specs=[
            pl.BlockSpec((NC, BM, D), lambda i: (0, i, 0)),
            pl.BlockSpec((BM, D), lambda i: (i, 0)),
            pl.BlockSpec((BM,), lambda i: (i,)),
            pl.BlockSpec((BM,), lambda i: (i,)),
            pl.BlockSpec((D,), lambda i: (0,)),
        ],
        out_specs=pl.BlockSpec((BM, D), lambda i: (i, 0)),
        out_shape=jax.ShapeDtypeStruct((N_PAD, D), jnp.float32),
    )(p2, h2, dis, inv, b2)


def kernel(x, edge_index, W1, b1, W2, b2):
    e = edge_index.shape[1]
    src = edge_index[0]
    dst = edge_index[1]

    pad_real = E_SC - e
    padv = jnp.full((pad_real,), N_NODES, jnp.int32)
    src_r = jnp.concatenate([src, padv]).reshape(NW, C, K)
    dst_r = jnp.concatenate([dst, padv]).reshape(NW, C, K)
    pad_chunks = jnp.full((NW, CP - C, K), N_NODES, jnp.int32)
    src_p = jnp.concatenate([src_r, pad_chunks], axis=1)
    dst_p = jnp.concatenate([dst_r, pad_chunks], axis=1)
    idx_comb = jnp.stack([src_p, dst_p], axis=2)

    pad_h = NW * EPW_H - e
    dst_h = jnp.concatenate(
        [dst, jnp.full((pad_h,), N_NODES, jnp.int32)]).reshape(NW, EPW_H)

    x_p = jnp.zeros((N_PAD, D), jnp.float32).at[:N_NODES].set(x)

    hists = _sc_degree_hist(dst_h)
    h1, hs1, dis, inv_deg = _tc_pre(hists, x_p, W1)
    p1 = _sc_edge_agg(hs1, idx_comb)
    h2, hs2 = _tc_mid(p1, h1, dis, inv_deg, b1, W2)
    p2 = _sc_edge_agg(hs2, idx_comb)
    out_p = _tc_post(p2, h2, dis, inv_deg, b2)
    return out_p[:N_NODES]
